# baseline (speedup 1.0000x reference)
import jax
import jax.numpy as jnp
from jax import lax
from jax.experimental import pallas as pl
from jax.experimental.pallas import tpu as pltpu
from jax.experimental.pallas import tpu_sc as plsc

NC = 2
NS = 16
CH = 128


def _make_sc_agg(n_feat, n_acc, k_chunks):
  rows = n_acc // NS
  mesh = plsc.VectorSubcoreMesh(core_axis_name="c", subcore_axis_name="s")

  assert k_chunks % 4 == 0
  half = k_chunks // 2

  def body(feat, src3, dst3, z2d, aggp, src_v, dst_v, buf_a, buf_b, acc,
           sem_a, sem_b):
    c = lax.axis_index("c")
    s = lax.axis_index("s")
    base = s * rows

    pltpu.sync_copy(z2d, acc.at[pl.ds(base, rows)])
    plsc.subcore_barrier()

    for p in range(2):
      pltpu.sync_copy(src3.at[c, s, pl.ds(p * half, half)], src_v)
      pltpu.sync_copy(dst3.at[c, s, pl.ds(p * half, half)], dst_v)

      pltpu.async_copy(feat.at[src_v.at[0]], buf_a, sem_a)

      def step(i, carry):
        ja = 2 * i
        jb = 2 * i + 1
        jn = jnp.minimum(jb + 1, half - 1)
        pltpu.make_async_copy(feat.at[src_v.at[ja]], buf_a, sem_a).wait()
        pltpu.async_copy(feat.at[src_v.at[jb]], buf_b, sem_b)
        pltpu.sync_copy(buf_a, acc.at[dst_v.at[ja]], add=True)
        pltpu.make_async_copy(feat.at[src_v.at[jb]], buf_b, sem_b).wait()
        pltpu.async_copy(feat.at[src_v.at[jn]], buf_a, sem_a)
        pltpu.sync_copy(buf_b, acc.at[dst_v.at[jb]], add=True)
        return carry

      lax.fori_loop(0, half // 2, step, 0)
      pltpu.make_async_copy(feat.at[src_v.at[half - 1]], buf_a, sem_a).wait()

    plsc.subcore_barrier()
    pltpu.sync_copy(acc.at[pl.ds(base, rows)], aggp.at[c, pl.ds(base, rows)])

  return pl.kernel(
      body,
      out_type=jax.ShapeDtypeStruct((NC, n_acc, n_feat), jnp.float32),
      mesh=mesh,
      scratch_types=[
          pltpu.VMEM((k_chunks // 2, CH), jnp.int32),
          pltpu.VMEM((k_chunks // 2, CH), jnp.int32),
          pltpu.VMEM((CH, n_feat), jnp.float32),
          pltpu.VMEM((CH, n_feat), jnp.float32),
          pltpu.VMEM_SHARED((n_acc, n_feat), jnp.float32),
          pltpu.SemaphoreType.DMA,
          pltpu.SemaphoreType.DMA,
      ])


def _make_sc_deg(n_acc, k_chunks):
  rows = n_acc // NS
  mesh = plsc.VectorSubcoreMesh(core_axis_name="c", subcore_axis_name="s")

  def body(dst3, z1d, ones1, degp, dst_v, ones_v, stage_v, degsh):
    c = lax.axis_index("c")
    s = lax.axis_index("s")
    base = s * rows

    pltpu.sync_copy(z1d, stage_v)
    pltpu.sync_copy(stage_v, degsh.at[pl.ds(base, rows)])
    pltpu.sync_copy(ones1, ones_v)
    plsc.subcore_barrier()

    pltpu.sync_copy(dst3.at[c, s], dst_v)

    def step(j, carry):
      pltpu.sync_copy(ones_v, degsh.at[dst_v.at[j]], add=True)
      return carry

    lax.fori_loop(0, k_chunks, step, 0)

    plsc.subcore_barrier()
    pltpu.sync_copy(degsh.at[pl.ds(base, rows)], stage_v)
    pltpu.sync_copy(stage_v, degp.at[pl.ds(c * n_acc + base, rows)])

  return pl.kernel(
      body,
      out_type=jax.ShapeDtypeStruct((NC * n_acc,), jnp.float32),
      mesh=mesh,
      scratch_types=[
          pltpu.VMEM((k_chunks, CH), jnp.int32),
          pltpu.VMEM((CH,), jnp.float32),
          pltpu.VMEM((rows,), jnp.float32),
          pltpu.VMEM_SHARED((n_acc,), jnp.float32),
      ])


def _tc_layer1(aggp_ref, degp_ref, x_ref, wn_ref, ws_ref, b_ref, out_ref):
  agg = aggp_ref[0] + aggp_ref[1]
  deg = degp_ref[0] + degp_ref[1]
  inv = 1.0 / jnp.maximum(deg, 1.0)
  hn = jnp.dot(agg, wn_ref[...], preferred_element_type=jnp.float32) * inv
  hs = jnp.dot(x_ref[...], ws_ref[...], preferred_element_type=jnp.float32)
  out_ref[...] = jax.nn.relu(hn + hs + b_ref[...])


def _tc_layer2(aggp_ref, degp_ref, h_ref, wn_ref, ws_ref, b_ref,
               wc_ref, bc_ref, out_ref):
  agg = aggp_ref[0] + aggp_ref[1]
  deg = degp_ref[0] + degp_ref[1]
  inv = 1.0 / jnp.maximum(deg, 1.0)
  hn = jnp.dot(agg, wn_ref[...], preferred_element_type=jnp.float32) * inv
  hs = jnp.dot(h_ref[...], ws_ref[...], preferred_element_type=jnp.float32)
  h2 = jax.nn.relu(hn + hs + b_ref[...])
  out_ref[...] = jnp.dot(h2, wc_ref[...],
                         preferred_element_type=jnp.float32) + bc_ref[...]


@jax.jit
def kernel(x, edge_index, W1_neigh, W1_self, b1, W2_neigh, W2_self, b2, Wc, bc):
  n, n_feat = x.shape
  e = edge_index.shape[1]
  n_cls = Wc.shape[0]

  k_chunks = -(-e // (NC * NS * CH))
  k_chunks = -(-k_chunks // 4) * 4
  e_pad = NC * NS * k_chunks * CH
  n_acc = -(-(n + 1) // (NS * 8)) * (NS * 8)
  rows = n_acc // NS

  src = edge_index[0].astype(jnp.int32)
  dst = edge_index[1].astype(jnp.int32)
  pad = e_pad - e
  src3 = jnp.concatenate([src, jnp.zeros((pad,), jnp.int32)]) \
      .reshape(NC, NS, k_chunks, CH)
  dst3 = jnp.concatenate([dst, jnp.full((pad,), n, jnp.int32)]) \
      .reshape(NC, NS, k_chunks, CH)

  z2d = jnp.zeros((rows, n_feat), jnp.float32)
  z1d = jnp.zeros((rows,), jnp.float32)
  ones1 = jnp.ones((CH,), jnp.float32)

  sc_agg = _make_sc_agg(n_feat, n_acc, k_chunks)
  sc_deg = _make_sc_deg(n_acc, k_chunks)

  degp = sc_deg(dst3, z1d, ones1).reshape(NC, n_acc, 1)
  aggp1 = sc_agg(x, src3, dst3, z2d)

  blk = 400
  grid = (n // blk,)
  layer1 = pl.pallas_call(
      _tc_layer1,
      grid=grid,
      in_specs=[
          pl.BlockSpec((NC, blk, n_feat), lambda i: (0, i, 0)),
          pl.BlockSpec((NC, blk, 1), lambda i: (0, i, 0)),
          pl.BlockSpec((blk, n_feat), lambda i: (i, 0)),
          pl.BlockSpec((n_feat, n_feat), lambda i: (0, 0)),
          pl.BlockSpec((n_feat, n_feat), lambda i: (0, 0)),
          pl.BlockSpec((1, n_feat), lambda i: (0, 0)),
      ],
      out_specs=pl.BlockSpec((blk, n_feat), lambda i: (i, 0)),
      out_shape=jax.ShapeDtypeStruct((n, n_feat), jnp.float32),
  )
  h = layer1(aggp1, degp, x, W1_neigh.T, W1_self.T, b1.reshape(1, -1))

  aggp2 = sc_agg(h, src3, dst3, z2d)

  layer2 = pl.pallas_call(
      _tc_layer2,
      grid=grid,
      in_specs=[
          pl.BlockSpec((NC, blk, n_feat), lambda i: (0, i, 0)),
          pl.BlockSpec((NC, blk, 1), lambda i: (0, i, 0)),
          pl.BlockSpec((blk, n_feat), lambda i: (i, 0)),
          pl.BlockSpec((n_feat, n_feat), lambda i: (0, 0)),
          pl.BlockSpec((n_feat, n_feat), lambda i: (0, 0)),
          pl.BlockSpec((1, n_feat), lambda i: (0, 0)),
          pl.BlockSpec((n_feat, n_cls), lambda i: (0, 0)),
          pl.BlockSpec((1, n_cls), lambda i: (0, 0)),
      ],
      out_specs=pl.BlockSpec((blk, n_cls), lambda i: (i, 0)),
      out_shape=jax.ShapeDtypeStruct((n, n_cls), jnp.float32),
  )
  out = layer2(aggp2, degp, h, W2_neigh.T, W2_self.T, b2.reshape(1, -1),
               Wc.T, bc.reshape(1, -1))
  return out


# speedup vs baseline: 1.4736x; 1.4736x over previous
import jax
import jax.numpy as jnp
from jax import lax
from jax.experimental import pallas as pl
from jax.experimental.pallas import tpu as pltpu
from jax.experimental.pallas import tpu_sc as plsc

NC = 2
NS = 16
CH = 128


def _make_sc_agg(n_feat, n_acc, k_chunks):
  rows = n_acc // NS
  mesh = plsc.VectorSubcoreMesh(core_axis_name="c", subcore_axis_name="s")

  def body(feat, src3, dst3, z2d, aggp, src_v, dst_v, rows_v, acc):
    c = lax.axis_index("c")
    s = lax.axis_index("s")
    base = s * rows

    pltpu.sync_copy(z2d, acc.at[pl.ds(base, rows)])
    plsc.subcore_barrier()

    pltpu.sync_copy(src3.at[c, s], src_v)
    pltpu.sync_copy(dst3.at[c, s], dst_v)

    def step(j, carry):
      pltpu.sync_copy(feat.at[src_v.at[j]], rows_v)
      pltpu.sync_copy(rows_v, acc.at[dst_v.at[j]], add=True)
      return carry

    lax.fori_loop(0, k_chunks, step, 0)

    plsc.subcore_barrier()
    pltpu.sync_copy(acc.at[pl.ds(base, rows)], aggp.at[c, pl.ds(base, rows)])

  return pl.kernel(
      body,
      out_type=jax.ShapeDtypeStruct((NC, n_acc, n_feat), jnp.float32),
      mesh=mesh,
      scratch_types=[
          pltpu.VMEM((k_chunks, CH), jnp.int32),
          pltpu.VMEM((k_chunks, CH), jnp.int32),
          pltpu.VMEM((CH, n_feat), jnp.float32),
          pltpu.VMEM_SHARED((n_acc, n_feat), jnp.float32),
      ])


def _make_sc_deg(n_acc, k_chunks):
  rows = n_acc // NS
  mesh = plsc.VectorSubcoreMesh(core_axis_name="c", subcore_axis_name="s")

  def body(dst3, z1d, ones1, degp, dst_v, ones_v, stage_v, degsh):
    c = lax.axis_index("c")
    s = lax.axis_index("s")
    base = s * rows

    pltpu.sync_copy(z1d, stage_v)
    pltpu.sync_copy(stage_v, degsh.at[pl.ds(base, rows)])
    pltpu.sync_copy(ones1, ones_v)
    plsc.subcore_barrier()

    pltpu.sync_copy(dst3.at[c, s], dst_v)

    def step(j, carry):
      pltpu.sync_copy(ones_v, degsh.at[dst_v.at[j]], add=True)
      return carry

    lax.fori_loop(0, k_chunks, step, 0)

    plsc.subcore_barrier()
    pltpu.sync_copy(degsh.at[pl.ds(base, rows)], stage_v)
    pltpu.sync_copy(stage_v, degp.at[pl.ds(c * n_acc + base, rows)])

  return pl.kernel(
      body,
      out_type=jax.ShapeDtypeStruct((NC * n_acc,), jnp.float32),
      mesh=mesh,
      scratch_types=[
          pltpu.VMEM((k_chunks, CH), jnp.int32),
          pltpu.VMEM((CH,), jnp.float32),
          pltpu.VMEM((rows,), jnp.float32),
          pltpu.VMEM_SHARED((n_acc,), jnp.float32),
      ])


def _tc_layer1(aggp_ref, degp_ref, x_ref, wn_ref, ws_ref, b_ref, out_ref):
  agg = aggp_ref[0] + aggp_ref[1]
  deg = degp_ref[0] + degp_ref[1]
  inv = 1.0 / jnp.maximum(deg, 1.0)
  hn = jnp.dot(agg, wn_ref[...], preferred_element_type=jnp.float32) * inv
  hs = jnp.dot(x_ref[...], ws_ref[...], preferred_element_type=jnp.float32)
  out_ref[...] = jax.nn.relu(hn + hs + b_ref[...])


def _tc_layer2(aggp_ref, degp_ref, h_ref, wn_ref, ws_ref, b_ref,
               wc_ref, bc_ref, out_ref):
  agg = aggp_ref[0] + aggp_ref[1]
  deg = degp_ref[0] + degp_ref[1]
  inv = 1.0 / jnp.maximum(deg, 1.0)
  hn = jnp.dot(agg, wn_ref[...], preferred_element_type=jnp.float32) * inv
  hs = jnp.dot(h_ref[...], ws_ref[...], preferred_element_type=jnp.float32)
  h2 = jax.nn.relu(hn + hs + b_ref[...])
  out_ref[...] = jnp.dot(h2, wc_ref[...],
                         preferred_element_type=jnp.float32) + bc_ref[...]


@jax.jit
def kernel(x, edge_index, W1_neigh, W1_self, b1, W2_neigh, W2_self, b2, Wc, bc):
  n, n_feat = x.shape
  e = edge_index.shape[1]
  n_cls = Wc.shape[0]

  k_chunks = -(-e // (NC * NS * CH))
  e_pad = NC * NS * k_chunks * CH
  n_acc = -(-(n + 1) // (NS * 8)) * (NS * 8)
  rows = n_acc // NS

  src = edge_index[0].astype(jnp.int32)
  dst = edge_index[1].astype(jnp.int32)
  pad = e_pad - e
  src3 = jnp.concatenate([src, jnp.zeros((pad,), jnp.int32)]) \
      .reshape(NC, NS, k_chunks, CH)
  dst3 = jnp.concatenate([dst, jnp.full((pad,), n, jnp.int32)]) \
      .reshape(NC, NS, k_chunks, CH)

  z2d = jnp.zeros((rows, n_feat), jnp.float32)
  z1d = jnp.zeros((rows,), jnp.float32)
  ones1 = jnp.ones((CH,), jnp.float32)

  sc_agg = _make_sc_agg(n_feat, n_acc, k_chunks)
  sc_deg = _make_sc_deg(n_acc, k_chunks)

  degp = sc_deg(dst3, z1d, ones1).reshape(NC, n_acc, 1)
  aggp1 = sc_agg(x, src3, dst3, z2d)

  blk = 400
  grid = (n // blk,)
  layer1 = pl.pallas_call(
      _tc_layer1,
      grid=grid,
      in_specs=[
          pl.BlockSpec((NC, blk, n_feat), lambda i: (0, i, 0)),
          pl.BlockSpec((NC, blk, 1), lambda i: (0, i, 0)),
          pl.BlockSpec((blk, n_feat), lambda i: (i, 0)),
          pl.BlockSpec((n_feat, n_feat), lambda i: (0, 0)),
          pl.BlockSpec((n_feat, n_feat), lambda i: (0, 0)),
          pl.BlockSpec((1, n_feat), lambda i: (0, 0)),
      ],
      out_specs=pl.BlockSpec((blk, n_feat), lambda i: (i, 0)),
      out_shape=jax.ShapeDtypeStruct((n, n_feat), jnp.float32),
  )
  h = layer1(aggp1, degp, x, W1_neigh.T, W1_self.T, b1.reshape(1, -1))

  aggp2 = sc_agg(h, src3, dst3, z2d)

  layer2 = pl.pallas_call(
      _tc_layer2,
      grid=grid,
      in_specs=[
          pl.BlockSpec((NC, blk, n_feat), lambda i: (0, i, 0)),
          pl.BlockSpec((NC, blk, 1), lambda i: (0, i, 0)),
          pl.BlockSpec((blk, n_feat), lambda i: (i, 0)),
          pl.BlockSpec((n_feat, n_feat), lambda i: (0, 0)),
          pl.BlockSpec((n_feat, n_feat), lambda i: (0, 0)),
          pl.BlockSpec((1, n_feat), lambda i: (0, 0)),
          pl.BlockSpec((n_feat, n_cls), lambda i: (0, 0)),
          pl.BlockSpec((1, n_cls), lambda i: (0, 0)),
      ],
      out_specs=pl.BlockSpec((blk, n_cls), lambda i: (i, 0)),
      out_shape=jax.ShapeDtypeStruct((n, n_cls), jnp.float32),
  )
  out = layer2(aggp2, degp, h, W2_neigh.T, W2_self.T, b2.reshape(1, -1),
               Wc.T, bc.reshape(1, -1))
  return out


# speedup vs baseline: 1.4773x; 1.0025x over previous
import jax
import jax.numpy as jnp
from jax import lax
from jax.experimental import pallas as pl
from jax.experimental.pallas import tpu as pltpu
from jax.experimental.pallas import tpu_sc as plsc

NC = 2
NS = 16
CH = 128


def _make_sc_agg(n_feat, n_acc, k_chunks):
  rows = n_acc // NS
  mesh = plsc.VectorSubcoreMesh(core_axis_name="c", subcore_axis_name="s")

  def body(feat, src3, dst3, z2d, aggp, src_v, dst_v, rows_v, acc):
    c = lax.axis_index("c")
    s = lax.axis_index("s")
    base = s * rows

    pltpu.sync_copy(z2d, acc.at[pl.ds(base, rows)])
    plsc.subcore_barrier()

    pltpu.sync_copy(src3.at[c, s], src_v)
    pltpu.sync_copy(dst3.at[c, s], dst_v)

    def step(j, carry):
      pltpu.sync_copy(feat.at[src_v.at[j]], rows_v)
      pltpu.sync_copy(rows_v, acc.at[dst_v.at[j]], add=True)
      return carry

    lax.fori_loop(0, k_chunks, step, 0)

    plsc.subcore_barrier()
    pltpu.sync_copy(acc.at[pl.ds(base, rows)], aggp.at[c, pl.ds(base, rows)])

  return pl.kernel(
      body,
      out_type=jax.ShapeDtypeStruct((NC, n_acc, n_feat), jnp.float32),
      mesh=mesh,
      scratch_types=[
          pltpu.VMEM((k_chunks, CH), jnp.int32),
          pltpu.VMEM((k_chunks, CH), jnp.int32),
          pltpu.VMEM((CH, n_feat), jnp.float32),
          pltpu.VMEM_SHARED((n_acc, n_feat), jnp.float32),
      ])


def _make_sc_deg(n_acc, k_chunks):
  rows = n_acc // NS
  mesh = plsc.VectorSubcoreMesh(core_axis_name="c", subcore_axis_name="s")

  def body(dst3, z1d, ones1, degp, dst_v, ones_v, stage_v, degsh):
    c = lax.axis_index("c")
    s = lax.axis_index("s")
    base = s * rows

    pltpu.sync_copy(z1d, stage_v)
    pltpu.sync_copy(stage_v, degsh.at[pl.ds(base, rows)])
    pltpu.sync_copy(ones1, ones_v)
    plsc.subcore_barrier()

    pltpu.sync_copy(dst3.at[c, s], dst_v)

    def step(j, carry):
      pltpu.sync_copy(ones_v, degsh.at[dst_v.at[j]], add=True)
      return carry

    lax.fori_loop(0, k_chunks, step, 0)

    plsc.subcore_barrier()
    pltpu.sync_copy(degsh.at[pl.ds(base, rows)], stage_v)
    pltpu.sync_copy(stage_v, degp.at[pl.ds(c * n_acc + base, rows)])

  return pl.kernel(
      body,
      out_type=jax.ShapeDtypeStruct((NC * n_acc,), jnp.float32),
      mesh=mesh,
      scratch_types=[
          pltpu.VMEM((k_chunks, CH), jnp.int32),
          pltpu.VMEM((CH,), jnp.float32),
          pltpu.VMEM((rows,), jnp.float32),
          pltpu.VMEM_SHARED((n_acc,), jnp.float32),
      ])


def _tc_layer1(aggp_ref, degp_ref, x_ref, wn_ref, ws_ref, b_ref, out_ref):
  agg = aggp_ref[0] + aggp_ref[1]
  deg = degp_ref[0] + degp_ref[1]
  inv = 1.0 / jnp.maximum(deg, 1.0)
  hn = jnp.dot(agg, wn_ref[...], preferred_element_type=jnp.float32) * inv
  hs = jnp.dot(x_ref[...], ws_ref[...], preferred_element_type=jnp.float32)
  out_ref[...] = jax.nn.relu(hn + hs + b_ref[...])


def _tc_layer2(aggp_ref, degp_ref, h_ref, wn_ref, ws_ref, b_ref,
               wc_ref, bc_ref, out_ref):
  agg = aggp_ref[0] + aggp_ref[1]
  deg = degp_ref[0] + degp_ref[1]
  inv = 1.0 / jnp.maximum(deg, 1.0)
  hn = jnp.dot(agg, wn_ref[...], preferred_element_type=jnp.float32) * inv
  hs = jnp.dot(h_ref[...], ws_ref[...], preferred_element_type=jnp.float32)
  h2 = jax.nn.relu(hn + hs + b_ref[...])
  out_ref[...] = jnp.dot(h2, wc_ref[...],
                         preferred_element_type=jnp.float32) + bc_ref[...]


@jax.jit
def kernel(x, edge_index, W1_neigh, W1_self, b1, W2_neigh, W2_self, b2, Wc, bc):
  n, n_feat = x.shape
  e = edge_index.shape[1]
  n_cls = Wc.shape[0]

  k_chunks = -(-e // (NC * NS * CH))
  e_pad = NC * NS * k_chunks * CH
  n_acc = -(-(n + 1) // (NS * 8)) * (NS * 8)
  rows = n_acc // NS

  src = edge_index[0].astype(jnp.int32)
  dst = edge_index[1].astype(jnp.int32)
  pad = e_pad - e
  src3 = jnp.concatenate([src, jnp.zeros((pad,), jnp.int32)]) \
      .reshape(NC, NS, k_chunks, CH)
  pad_dst = n + (jnp.arange(pad, dtype=jnp.int32) % (n_acc - n))
  dst3 = jnp.concatenate([dst, pad_dst]).reshape(NC, NS, k_chunks, CH)

  z2d = jnp.zeros((rows, n_feat), jnp.float32)
  z1d = jnp.zeros((rows,), jnp.float32)
  ones1 = jnp.ones((CH,), jnp.float32)

  sc_agg = _make_sc_agg(n_feat, n_acc, k_chunks)
  sc_deg = _make_sc_deg(n_acc, k_chunks)

  degp = sc_deg(dst3, z1d, ones1).reshape(NC, n_acc, 1)
  aggp1 = sc_agg(x, src3, dst3, z2d)

  blk = 400
  grid = (n // blk,)
  layer1 = pl.pallas_call(
      _tc_layer1,
      grid=grid,
      in_specs=[
          pl.BlockSpec((NC, blk, n_feat), lambda i: (0, i, 0)),
          pl.BlockSpec((NC, blk, 1), lambda i: (0, i, 0)),
          pl.BlockSpec((blk, n_feat), lambda i: (i, 0)),
          pl.BlockSpec((n_feat, n_feat), lambda i: (0, 0)),
          pl.BlockSpec((n_feat, n_feat), lambda i: (0, 0)),
          pl.BlockSpec((1, n_feat), lambda i: (0, 0)),
      ],
      out_specs=pl.BlockSpec((blk, n_feat), lambda i: (i, 0)),
      out_shape=jax.ShapeDtypeStruct((n, n_feat), jnp.float32),
  )
  h = layer1(aggp1, degp, x, W1_neigh.T, W1_self.T, b1.reshape(1, -1))

  aggp2 = sc_agg(h, src3, dst3, z2d)

  layer2 = pl.pallas_call(
      _tc_layer2,
      grid=grid,
      in_specs=[
          pl.BlockSpec((NC, blk, n_feat), lambda i: (0, i, 0)),
          pl.BlockSpec((NC, blk, 1), lambda i: (0, i, 0)),
          pl.BlockSpec((blk, n_feat), lambda i: (i, 0)),
          pl.BlockSpec((n_feat, n_feat), lambda i: (0, 0)),
          pl.BlockSpec((n_feat, n_feat), lambda i: (0, 0)),
          pl.BlockSpec((1, n_feat), lambda i: (0, 0)),
          pl.BlockSpec((n_feat, n_cls), lambda i: (0, 0)),
          pl.BlockSpec((1, n_cls), lambda i: (0, 0)),
      ],
      out_specs=pl.BlockSpec((blk, n_cls), lambda i: (i, 0)),
      out_shape=jax.ShapeDtypeStruct((n, n_cls), jnp.float32),
  )
  out = layer2(aggp2, degp, h, W2_neigh.T, W2_self.T, b2.reshape(1, -1),
               Wc.T, bc.reshape(1, -1))
  return out


# speedup vs baseline: 2.1122x; 1.4298x over previous
import jax
import jax.numpy as jnp
from jax import lax
from jax.experimental import pallas as pl
from jax.experimental.pallas import tpu as pltpu
from jax.experimental.pallas import tpu_sc as plsc

NC = 2
NS = 16
CH = 128


def _make_sc_agg(n_feat, n_acc, k0, k1):
  rows = n_acc // NS
  kmax = max(k0, k1)
  mesh = plsc.VectorSubcoreMesh(core_axis_name="c", subcore_axis_name="s")

  def body(feat, src3, dst3, z2d, aggp, src_v, dst_v, rows_v, acc):
    c = lax.axis_index("c")
    s = lax.axis_index("s")
    base = s * rows
    kc = jnp.where(c == 0, k0, k1)

    pltpu.sync_copy(z2d, acc.at[pl.ds(base, rows)])
    plsc.subcore_barrier()

    pltpu.sync_copy(src3.at[c, s], src_v)
    pltpu.sync_copy(dst3.at[c, s], dst_v)

    def step(j, carry):
      pltpu.sync_copy(feat.at[src_v.at[j]], rows_v)
      pltpu.sync_copy(rows_v, acc.at[dst_v.at[j]], add=True)
      return carry

    lax.fori_loop(0, kc, step, 0)

    plsc.subcore_barrier()
    pltpu.sync_copy(acc.at[pl.ds(base, rows)], aggp.at[c, pl.ds(base, rows)])

  return pl.kernel(
      body,
      out_type=jax.ShapeDtypeStruct((NC, n_acc, n_feat), jnp.float32),
      mesh=mesh,
      scratch_types=[
          pltpu.VMEM((kmax, CH), jnp.int32),
          pltpu.VMEM((kmax, CH), jnp.int32),
          pltpu.VMEM((CH, n_feat), jnp.float32),
          pltpu.VMEM_SHARED((n_acc, n_feat), jnp.float32),
      ])


def _make_sc_deg(n_acc, k0, k1):
  rows = n_acc // NS
  kmax = max(k0, k1)
  mesh = plsc.VectorSubcoreMesh(core_axis_name="c", subcore_axis_name="s")

  def body(dst3, z1d, ones1, degp, dst_v, ones_v, stage_v, degsh):
    c = lax.axis_index("c")
    s = lax.axis_index("s")
    base = s * rows
    kc = jnp.where(c == 0, k0, k1)

    pltpu.sync_copy(z1d, stage_v)
    pltpu.sync_copy(stage_v, degsh.at[pl.ds(base, rows)])
    pltpu.sync_copy(ones1, ones_v)
    plsc.subcore_barrier()

    pltpu.sync_copy(dst3.at[c, s], dst_v)

    def step(j, carry):
      pltpu.sync_copy(ones_v, degsh.at[dst_v.at[j]], add=True)
      return carry

    lax.fori_loop(0, kc, step, 0)

    plsc.subcore_barrier()
    pltpu.sync_copy(degsh.at[pl.ds(base, rows)], stage_v)
    pltpu.sync_copy(stage_v, degp.at[pl.ds(c * n_acc + base, rows)])

  return pl.kernel(
      body,
      out_type=jax.ShapeDtypeStruct((NC * n_acc,), jnp.float32),
      mesh=mesh,
      scratch_types=[
          pltpu.VMEM((kmax, CH), jnp.int32),
          pltpu.VMEM((CH,), jnp.float32),
          pltpu.VMEM((rows,), jnp.float32),
          pltpu.VMEM_SHARED((n_acc,), jnp.float32),
      ])


def _tc_layer1(aggp_ref, degp_ref, x_ref, wn_ref, ws_ref, b_ref, out_ref):
  agg = aggp_ref[0] + aggp_ref[1]
  deg = degp_ref[0] + degp_ref[1]
  inv = 1.0 / jnp.maximum(deg, 1.0)
  hn = jnp.dot(agg, wn_ref[...], preferred_element_type=jnp.float32) * inv
  hs = jnp.dot(x_ref[...], ws_ref[...], preferred_element_type=jnp.float32)
  out_ref[...] = jax.nn.relu(hn + hs + b_ref[...])


def _tc_layer2(aggp_ref, degp_ref, h_ref, wn_ref, ws_ref, b_ref,
               wc_ref, bc_ref, out_ref):
  agg = aggp_ref[0] + aggp_ref[1]
  deg = degp_ref[0] + degp_ref[1]
  inv = 1.0 / jnp.maximum(deg, 1.0)
  hn = jnp.dot(agg, wn_ref[...], preferred_element_type=jnp.float32) * inv
  hs = jnp.dot(h_ref[...], ws_ref[...], preferred_element_type=jnp.float32)
  h2 = jax.nn.relu(hn + hs + b_ref[...])
  out_ref[...] = jnp.dot(h2, wc_ref[...],
                         preferred_element_type=jnp.float32) + bc_ref[...]


@jax.jit
def kernel(x, edge_index, W1_neigh, W1_self, b1, W2_neigh, W2_self, b2, Wc, bc):
  n, n_feat = x.shape
  e = edge_index.shape[1]
  n_cls = Wc.shape[0]

  n_acc = -(-(n + 1) // (NS * 8)) * (NS * 8)
  rows = n_acc // NS

  n_chunks = -(-e // CH)
  c0 = int(round(n_chunks * 0.65 / NS)) * NS
  c1 = n_chunks - c0
  k0 = c0 // NS
  k1 = -(-c1 // NS)
  kmax = max(k0, k1)
  e_pad = (c0 + NS * k1) * CH

  src = edge_index[0].astype(jnp.int32)
  dst = edge_index[1].astype(jnp.int32)
  pad = e_pad - e
  pad_dst = n + (jnp.arange(pad, dtype=jnp.int32) % (n_acc - n))
  src_all = jnp.concatenate([src, jnp.zeros((pad,), jnp.int32)])
  dst_all = jnp.concatenate([dst, pad_dst])

  def to_slabs(a):
    s0 = a[:c0 * CH].reshape(NS, k0, CH)
    s1 = a[c0 * CH:].reshape(NS, k1, CH)
    s0 = jnp.pad(s0, ((0, 0), (0, kmax - k0), (0, 0)))
    s1 = jnp.pad(s1, ((0, 0), (0, kmax - k1), (0, 0)))
    return jnp.stack([s0, s1])

  src3 = to_slabs(src_all)
  dst3 = to_slabs(dst_all)

  z2d = jnp.zeros((rows, n_feat), jnp.float32)
  z1d = jnp.zeros((rows,), jnp.float32)
  ones1 = jnp.ones((CH,), jnp.float32)

  sc_agg = _make_sc_agg(n_feat, n_acc, k0, k1)
  sc_deg = _make_sc_deg(n_acc, k0, k1)

  degp = sc_deg(dst3, z1d, ones1).reshape(NC, n_acc, 1)
  aggp1 = sc_agg(x, src3, dst3, z2d)

  blk = 400
  grid = (n // blk,)
  layer1 = pl.pallas_call(
      _tc_layer1,
      grid=grid,
      in_specs=[
          pl.BlockSpec((NC, blk, n_feat), lambda i: (0, i, 0)),
          pl.BlockSpec((NC, blk, 1), lambda i: (0, i, 0)),
          pl.BlockSpec((blk, n_feat), lambda i: (i, 0)),
          pl.BlockSpec((n_feat, n_feat), lambda i: (0, 0)),
          pl.BlockSpec((n_feat, n_feat), lambda i: (0, 0)),
          pl.BlockSpec((1, n_feat), lambda i: (0, 0)),
      ],
      out_specs=pl.BlockSpec((blk, n_feat), lambda i: (i, 0)),
      out_shape=jax.ShapeDtypeStruct((n, n_feat), jnp.float32),
  )
  h = layer1(aggp1, degp, x, W1_neigh.T, W1_self.T, b1.reshape(1, -1))

  aggp2 = sc_agg(h, src3, dst3, z2d)

  layer2 = pl.pallas_call(
      _tc_layer2,
      grid=grid,
      in_specs=[
          pl.BlockSpec((NC, blk, n_feat), lambda i: (0, i, 0)),
          pl.BlockSpec((NC, blk, 1), lambda i: (0, i, 0)),
          pl.BlockSpec((blk, n_feat), lambda i: (i, 0)),
          pl.BlockSpec((n_feat, n_feat), lambda i: (0, 0)),
          pl.BlockSpec((n_feat, n_feat), lambda i: (0, 0)),
          pl.BlockSpec((1, n_feat), lambda i: (0, 0)),
          pl.BlockSpec((n_feat, n_cls), lambda i: (0, 0)),
          pl.BlockSpec((1, n_cls), lambda i: (0, 0)),
      ],
      out_specs=pl.BlockSpec((blk, n_cls), lambda i: (i, 0)),
      out_shape=jax.ShapeDtypeStruct((n, n_cls), jnp.float32),
  )
  out = layer2(aggp2, degp, h, W2_neigh.T, W2_self.T, b2.reshape(1, -1),
               Wc.T, bc.reshape(1, -1))
  return out


# speedup vs baseline: 2.3875x; 1.1303x over previous
import jax
import jax.numpy as jnp
from jax import lax
from jax.experimental import pallas as pl
from jax.experimental.pallas import tpu as pltpu
from jax.experimental.pallas import tpu_sc as plsc

NC = 2
NS = 16
CH = 128
F0 = 0.627


def _tile_count(c, s, nc0, nc1):
  return (jnp.where(c == 0, nc0, nc1) - s + NS - 1) // NS


def _make_sc_agg(n_feat, n_acc, nc0, nc1, kmax):
  rows = n_acc // NS
  mesh = plsc.VectorSubcoreMesh(core_axis_name="c", subcore_axis_name="s")

  def body(feat, src4, dst4, z2d, aggp, src_v, dst_v, rows_v, acc):
    c = lax.axis_index("c")
    s = lax.axis_index("s")
    base = s * rows
    kc = _tile_count(c, s, nc0, nc1)

    pltpu.sync_copy(z2d, acc.at[pl.ds(base, rows)])
    plsc.subcore_barrier()

    pltpu.sync_copy(src4.at[c, :, pl.ds(s * CH, CH)], src_v)
    pltpu.sync_copy(dst4.at[c, :, pl.ds(s * CH, CH)], dst_v)

    def step(j, carry):
      pltpu.sync_copy(feat.at[src_v.at[j]], rows_v)
      pltpu.sync_copy(rows_v, acc.at[dst_v.at[j]], add=True)
      return carry

    lax.fori_loop(0, kc, step, 0)

    plsc.subcore_barrier()
    pltpu.sync_copy(acc.at[pl.ds(base, rows)], aggp.at[c, pl.ds(base, rows)])

  return pl.kernel(
      body,
      out_type=jax.ShapeDtypeStruct((NC, n_acc, n_feat), jnp.float32),
      mesh=mesh,
      scratch_types=[
          pltpu.VMEM((kmax, CH), jnp.int32),
          pltpu.VMEM((kmax, CH), jnp.int32),
          pltpu.VMEM((CH, n_feat), jnp.float32),
          pltpu.VMEM_SHARED((n_acc, n_feat), jnp.float32),
      ])


def _make_sc_deg(n_acc, nc0, nc1, kmax):
  rows = n_acc // NS
  mesh = plsc.VectorSubcoreMesh(core_axis_name="c", subcore_axis_name="s")

  def body(dst4, z1d, ones1, degp, dst_v, ones_v, stage_v, degsh):
    c = lax.axis_index("c")
    s = lax.axis_index("s")
    base = s * rows
    kc = _tile_count(c, s, nc0, nc1)

    pltpu.sync_copy(z1d, stage_v)
    pltpu.sync_copy(stage_v, degsh.at[pl.ds(base, rows)])
    pltpu.sync_copy(ones1, ones_v)
    plsc.subcore_barrier()

    pltpu.sync_copy(dst4.at[c, :, pl.ds(s * CH, CH)], dst_v)

    def step(j, carry):
      pltpu.sync_copy(ones_v, degsh.at[dst_v.at[j]], add=True)
      return carry

    lax.fori_loop(0, kc, step, 0)

    plsc.subcore_barrier()
    pltpu.sync_copy(degsh.at[pl.ds(base, rows)], stage_v)
    pltpu.sync_copy(stage_v, degp.at[pl.ds(c * n_acc + base, rows)])

  return pl.kernel(
      body,
      out_type=jax.ShapeDtypeStruct((NC * n_acc,), jnp.float32),
      mesh=mesh,
      scratch_types=[
          pltpu.VMEM((kmax, CH), jnp.int32),
          pltpu.VMEM((CH,), jnp.float32),
          pltpu.VMEM((rows,), jnp.float32),
          pltpu.VMEM_SHARED((n_acc,), jnp.float32),
      ])


def _tc_layer1(aggp_ref, degp_ref, x_ref, wn_ref, ws_ref, b_ref, out_ref):
  agg = aggp_ref[0] + aggp_ref[1]
  deg = degp_ref[0] + degp_ref[1]
  inv = 1.0 / jnp.maximum(deg, 1.0)
  hn = jnp.dot(agg, wn_ref[...], preferred_element_type=jnp.float32) * inv
  hs = jnp.dot(x_ref[...], ws_ref[...], preferred_element_type=jnp.float32)
  out_ref[...] = jax.nn.relu(hn + hs + b_ref[...])


def _tc_layer2(aggp_ref, degp_ref, h_ref, wn_ref, ws_ref, b_ref,
               wc_ref, bc_ref, out_ref):
  agg = aggp_ref[0] + aggp_ref[1]
  deg = degp_ref[0] + degp_ref[1]
  inv = 1.0 / jnp.maximum(deg, 1.0)
  hn = jnp.dot(agg, wn_ref[...], preferred_element_type=jnp.float32) * inv
  hs = jnp.dot(h_ref[...], ws_ref[...], preferred_element_type=jnp.float32)
  h2 = jax.nn.relu(hn + hs + b_ref[...])
  out_ref[...] = jnp.dot(h2, wc_ref[...],
                         preferred_element_type=jnp.float32) + bc_ref[...]


@jax.jit
def kernel(x, edge_index, W1_neigh, W1_self, b1, W2_neigh, W2_self, b2, Wc, bc):
  n, n_feat = x.shape
  e = edge_index.shape[1]
  n_cls = Wc.shape[0]

  n_acc = -(-(n + 1) // (NS * 8)) * (NS * 8)
  rows = n_acc // NS

  n_chunks = -(-e // CH)
  kmax = int(round(n_chunks * F0 / NS))
  nc0 = kmax * NS
  nc1 = n_chunks - nc0
  assert 0 < nc1 <= nc0

  src = edge_index[0].astype(jnp.int32)
  dst = edge_index[1].astype(jnp.int32)
  pad = NC * NS * kmax * CH - e
  pad_dst = n + (jnp.arange(pad, dtype=jnp.int32) % (n_acc - n))
  src4 = jnp.concatenate([src, jnp.zeros((pad,), jnp.int32)]) \
      .reshape(NC, kmax, NS * CH)
  dst4 = jnp.concatenate([dst, pad_dst]).reshape(NC, kmax, NS * CH)

  z2d = jnp.zeros((rows, n_feat), jnp.float32)
  z1d = jnp.zeros((rows,), jnp.float32)
  ones1 = jnp.ones((CH,), jnp.float32)

  sc_agg = _make_sc_agg(n_feat, n_acc, nc0, nc1, kmax)
  sc_deg = _make_sc_deg(n_acc, nc0, nc1, kmax)

  degp = sc_deg(dst4, z1d, ones1).reshape(NC, n_acc, 1)
  aggp1 = sc_agg(x, src4, dst4, z2d)

  blk = 2000
  grid = (n // blk,)
  layer1 = pl.pallas_call(
      _tc_layer1,
      grid=grid,
      in_specs=[
          pl.BlockSpec((NC, blk, n_feat), lambda i: (0, i, 0)),
          pl.BlockSpec((NC, blk, 1), lambda i: (0, i, 0)),
          pl.BlockSpec((blk, n_feat), lambda i: (i, 0)),
          pl.BlockSpec((n_feat, n_feat), lambda i: (0, 0)),
          pl.BlockSpec((n_feat, n_feat), lambda i: (0, 0)),
          pl.BlockSpec((1, n_feat), lambda i: (0, 0)),
      ],
      out_specs=pl.BlockSpec((blk, n_feat), lambda i: (i, 0)),
      out_shape=jax.ShapeDtypeStruct((n, n_feat), jnp.float32),
  )
  h = layer1(aggp1, degp, x, W1_neigh.T, W1_self.T, b1.reshape(1, -1))

  aggp2 = sc_agg(h, src4, dst4, z2d)

  layer2 = pl.pallas_call(
      _tc_layer2,
      grid=grid,
      in_specs=[
          pl.BlockSpec((NC, blk, n_feat), lambda i: (0, i, 0)),
          pl.BlockSpec((NC, blk, 1), lambda i: (0, i, 0)),
          pl.BlockSpec((blk, n_feat), lambda i: (i, 0)),
          pl.BlockSpec((n_feat, n_feat), lambda i: (0, 0)),
          pl.BlockSpec((n_feat, n_feat), lambda i: (0, 0)),
          pl.BlockSpec((1, n_feat), lambda i: (0, 0)),
          pl.BlockSpec((n_feat, n_cls), lambda i: (0, 0)),
          pl.BlockSpec((1, n_cls), lambda i: (0, 0)),
      ],
      out_specs=pl.BlockSpec((blk, n_cls), lambda i: (i, 0)),
      out_shape=jax.ShapeDtypeStruct((n, n_cls), jnp.float32),
  )
  out = layer2(aggp2, degp, h, W2_neigh.T, W2_self.T, b2.reshape(1, -1),
               Wc.T, bc.reshape(1, -1))
  return out


# speedup vs baseline: 2.7708x; 1.1606x over previous
import jax
import jax.numpy as jnp
from jax import lax
from jax.experimental import pallas as pl
from jax.experimental.pallas import tpu as pltpu
from jax.experimental.pallas import tpu_sc as plsc

NC = 2
NS = 16
CH = 128
F0 = 0.52


def _tile_count(c, s, nc0, nc1):
  return (jnp.where(c == 0, nc0, nc1) - s + NS - 1) // NS


def _make_sc_agg(n_feat, n_acc, nc0, nc1, kmax):
  rows = n_acc // NS
  mesh = plsc.VectorSubcoreMesh(core_axis_name="c", subcore_axis_name="s")

  def body(feat, src4, dst4, z2d, aggp, src_v, dst_v, rows_v, acc):
    c = lax.axis_index("c")
    s = lax.axis_index("s")
    base = s * rows
    kc = _tile_count(c, s, nc0, nc1)

    pltpu.sync_copy(z2d, acc.at[pl.ds(base, rows)])
    plsc.subcore_barrier()

    pltpu.sync_copy(src4.at[c, :, pl.ds(s * CH, CH)], src_v)
    pltpu.sync_copy(dst4.at[c, :, pl.ds(s * CH, CH)], dst_v)

    def step(j, carry):
      pltpu.sync_copy(feat.at[src_v.at[j]], rows_v)
      pltpu.sync_copy(rows_v, acc.at[dst_v.at[j]], add=True)
      return carry

    lax.fori_loop(0, kc, step, 0)

    plsc.subcore_barrier()
    pltpu.sync_copy(acc.at[pl.ds(base, rows)], aggp.at[c, pl.ds(base, rows)])

  return pl.kernel(
      body,
      out_type=jax.ShapeDtypeStruct((NC, n_acc, n_feat), jnp.float32),
      mesh=mesh,
      scratch_types=[
          pltpu.VMEM((kmax, CH), jnp.int32),
          pltpu.VMEM((kmax, CH), jnp.int32),
          pltpu.VMEM((CH, n_feat), jnp.float32),
          pltpu.VMEM_SHARED((n_acc, n_feat), jnp.float32),
      ])


def _make_sc_deg(n_acc, nc0, nc1, kmax):
  rows = n_acc // NS
  mesh = plsc.VectorSubcoreMesh(core_axis_name="c", subcore_axis_name="s")

  def body(dst4, z1d, ones1, degp, dst_v, ones_v, stage_v, degsh):
    c = lax.axis_index("c")
    s = lax.axis_index("s")
    base = s * rows
    kc = _tile_count(c, s, nc0, nc1)

    pltpu.sync_copy(z1d, stage_v)
    pltpu.sync_copy(stage_v, degsh.at[pl.ds(base, rows)])
    pltpu.sync_copy(ones1, ones_v)
    plsc.subcore_barrier()

    pltpu.sync_copy(dst4.at[c, :, pl.ds(s * CH, CH)], dst_v)

    def step(j, carry):
      pltpu.sync_copy(ones_v, degsh.at[dst_v.at[j]], add=True)
      return carry

    lax.fori_loop(0, kc, step, 0)

    plsc.subcore_barrier()
    pltpu.sync_copy(degsh.at[pl.ds(base, rows)], stage_v)
    pltpu.sync_copy(stage_v, degp.at[pl.ds(c * n_acc + base, rows)])

  return pl.kernel(
      body,
      out_type=jax.ShapeDtypeStruct((NC * n_acc,), jnp.float32),
      mesh=mesh,
      scratch_types=[
          pltpu.VMEM((kmax, CH), jnp.int32),
          pltpu.VMEM((CH,), jnp.float32),
          pltpu.VMEM((rows,), jnp.float32),
          pltpu.VMEM_SHARED((n_acc,), jnp.float32),
      ])


def _tc_layer1(aggp_ref, degp_ref, x_ref, wn_ref, ws_ref, b_ref, out_ref):
  agg = aggp_ref[0] + aggp_ref[1]
  deg = degp_ref[0] + degp_ref[1]
  inv = 1.0 / jnp.maximum(deg, 1.0)
  hn = jnp.dot(agg, wn_ref[...], preferred_element_type=jnp.float32) * inv
  hs = jnp.dot(x_ref[...], ws_ref[...], preferred_element_type=jnp.float32)
  out_ref[...] = jax.nn.relu(hn + hs + b_ref[...])


def _tc_layer2(aggp_ref, degp_ref, h_ref, wn_ref, ws_ref, b_ref,
               wc_ref, bc_ref, out_ref):
  agg = aggp_ref[0] + aggp_ref[1]
  deg = degp_ref[0] + degp_ref[1]
  inv = 1.0 / jnp.maximum(deg, 1.0)
  hn = jnp.dot(agg, wn_ref[...], preferred_element_type=jnp.float32) * inv
  hs = jnp.dot(h_ref[...], ws_ref[...], preferred_element_type=jnp.float32)
  h2 = jax.nn.relu(hn + hs + b_ref[...])
  out_ref[...] = jnp.dot(h2, wc_ref[...],
                         preferred_element_type=jnp.float32) + bc_ref[...]


@jax.jit
def kernel(x, edge_index, W1_neigh, W1_self, b1, W2_neigh, W2_self, b2, Wc, bc):
  n, n_feat = x.shape
  e = edge_index.shape[1]
  n_cls = Wc.shape[0]

  n_acc = -(-(n + 1) // (NS * 8)) * (NS * 8)
  rows = n_acc // NS

  n_chunks = -(-e // CH)
  kmax = int(round(n_chunks * F0 / NS))
  nc0 = kmax * NS
  nc1 = n_chunks - nc0
  assert 0 < nc1 <= nc0

  src = edge_index[0].astype(jnp.int32)
  dst = edge_index[1].astype(jnp.int32)
  pad = NC * NS * kmax * CH - e
  pad_dst = n + (jnp.arange(pad, dtype=jnp.int32) % (n_acc - n))
  src4 = jnp.concatenate([src, jnp.zeros((pad,), jnp.int32)]) \
      .reshape(NC, kmax, NS * CH)
  dst4 = jnp.concatenate([dst, pad_dst]).reshape(NC, kmax, NS * CH)

  z2d = jnp.zeros((rows, n_feat), jnp.float32)
  z1d = jnp.zeros((rows,), jnp.float32)
  ones1 = jnp.ones((CH,), jnp.float32)

  sc_agg = _make_sc_agg(n_feat, n_acc, nc0, nc1, kmax)
  sc_deg = _make_sc_deg(n_acc, nc0, nc1, kmax)

  degp = sc_deg(dst4, z1d, ones1).reshape(NC, n_acc, 1)
  aggp1 = sc_agg(x, src4, dst4, z2d)

  blk = 2000
  grid = (n // blk,)
  layer1 = pl.pallas_call(
      _tc_layer1,
      grid=grid,
      in_specs=[
          pl.BlockSpec((NC, blk, n_feat), lambda i: (0, i, 0)),
          pl.BlockSpec((NC, blk, 1), lambda i: (0, i, 0)),
          pl.BlockSpec((blk, n_feat), lambda i: (i, 0)),
          pl.BlockSpec((n_feat, n_feat), lambda i: (0, 0)),
          pl.BlockSpec((n_feat, n_feat), lambda i: (0, 0)),
          pl.BlockSpec((1, n_feat), lambda i: (0, 0)),
      ],
      out_specs=pl.BlockSpec((blk, n_feat), lambda i: (i, 0)),
      out_shape=jax.ShapeDtypeStruct((n, n_feat), jnp.float32),
  )
  h = layer1(aggp1, degp, x, W1_neigh.T, W1_self.T, b1.reshape(1, -1))

  aggp2 = sc_agg(h, src4, dst4, z2d)

  layer2 = pl.pallas_call(
      _tc_layer2,
      grid=grid,
      in_specs=[
          pl.BlockSpec((NC, blk, n_feat), lambda i: (0, i, 0)),
          pl.BlockSpec((NC, blk, 1), lambda i: (0, i, 0)),
          pl.BlockSpec((blk, n_feat), lambda i: (i, 0)),
          pl.BlockSpec((n_feat, n_feat), lambda i: (0, 0)),
          pl.BlockSpec((n_feat, n_feat), lambda i: (0, 0)),
          pl.BlockSpec((1, n_feat), lambda i: (0, 0)),
          pl.BlockSpec((n_feat, n_cls), lambda i: (0, 0)),
          pl.BlockSpec((1, n_cls), lambda i: (0, 0)),
      ],
      out_specs=pl.BlockSpec((blk, n_cls), lambda i: (i, 0)),
      out_shape=jax.ShapeDtypeStruct((n, n_cls), jnp.float32),
  )
  out = layer2(aggp2, degp, h, W2_neigh.T, W2_self.T, b2.reshape(1, -1),
               Wc.T, bc.reshape(1, -1))
  return out
